# Initial kernel scaffold; baseline (speedup 1.0000x reference)
import jax
import jax.numpy as jnp
from jax.experimental import pallas as pl


def kernel(pos, params, z, edge_index, batch):
    raise NotImplementedError("write your pallas kernel here")


# speedup vs baseline: 1.7102x; 1.7102x over previous
import functools

import jax
import jax.numpy as jnp
import numpy as np
from jax import lax
from jax.experimental import pallas as pl
from jax.experimental.pallas import tpu as pltpu
from jax.experimental.pallas import tpu_sc as plsc

N = 10000
E = 320000
H = 128
G = 50
NG = 64
ZMAX = 100
_SHIFT = float(np.log(2.0))

NC, NS = 2, 16
NW = NC * NS
CH = 128
E_PAD = 323584
EPT = E_PAD // NW
NCH = EPT // CH
NP = 10240
DUMMY = N
BN = 128
NB = NP // BN
BE = 512
NEB = E_PAD // BE

_mesh = lambda: plsc.VectorSubcoreMesh(core_axis_name="c", subcore_axis_name="s")


def _ssp(x):
    return jnp.maximum(x, 0.0) + jnp.log1p(jnp.exp(-jnp.abs(x))) - _SHIFT


def _make_gather2(D, tc_tiling=True):
    @functools.partial(
        pl.kernel,
        out_type=(jax.ShapeDtypeStruct((E_PAD, D), jnp.float32),
                  jax.ShapeDtypeStruct((E_PAD, D), jnp.float32)),
        mesh=_mesh(),
        compiler_params=pltpu.CompilerParams(use_tc_tiling_on_sc=tc_tiling),
        scratch_types=[
            pltpu.VMEM((CH,), jnp.int32),
            pltpu.VMEM((CH,), jnp.int32),
            pltpu.VMEM((CH, D), jnp.float32),
            pltpu.VMEM((CH, D), jnp.float32),
            pltpu.SemaphoreType.DMA,
            pltpu.SemaphoreType.DMA,
        ],
    )
    def k(table, ia, ib, oa, ob, iva, ivb, bufa, bufb, sema, semb):
        c = lax.axis_index("c")
        s = lax.axis_index("s")
        wid = c * NS + s

        def body(j, carry):
            base = wid * EPT + j * CH
            pltpu.sync_copy(ia.at[pl.ds(base, CH)], iva)
            pltpu.sync_copy(ib.at[pl.ds(base, CH)], ivb)
            ca = pltpu.async_copy(table.at[iva], bufa, sema)
            cb = pltpu.async_copy(table.at[ivb], bufb, semb)
            ca.wait()
            cb.wait()
            pltpu.sync_copy(bufa, oa.at[pl.ds(base, CH)])
            pltpu.sync_copy(bufb, ob.at[pl.ds(base, CH)])
            return carry

        lax.fori_loop(0, NCH, body, 0)

    return k


def _make_scatter():
    RPT = NP // NS

    @functools.partial(
        pl.kernel,
        out_type=jax.ShapeDtypeStruct((2 * NP, H), jnp.float32),
        mesh=_mesh(),
        scratch_types=[
            pltpu.VMEM((CH,), jnp.int32),
            pltpu.VMEM((CH, H), jnp.float32),
            pltpu.VMEM_SHARED((NP, H), jnp.float32),
        ],
    )
    def k(msg, dstix, zblk, out, idxv, bufm, acc):
        c = lax.axis_index("c")
        s = lax.axis_index("s")
        wid = c * NS + s

        pltpu.sync_copy(zblk, bufm)
        for t in range(RPT // CH):
            pltpu.sync_copy(bufm, acc.at[pl.ds(s * RPT + t * CH, CH)])
        plsc.subcore_barrier()

        def body(j, carry):
            base = wid * EPT + j * CH
            pltpu.sync_copy(dstix.at[pl.ds(base, CH)], idxv)
            pltpu.sync_copy(msg.at[pl.ds(base, CH)], bufm)
            pltpu.sync_copy(bufm, acc.at[idxv], add=True)
            return carry

        lax.fori_loop(0, NCH, body, 0)
        plsc.subcore_barrier()

        for t in range(RPT // CH):
            r0 = s * RPT + t * CH
            pltpu.sync_copy(acc.at[pl.ds(r0, CH)], bufm)
            pltpu.sync_copy(bufm, out.at[pl.ds(c * NP + r0, CH)])

    return k


_gather_pos = _make_gather2(16, tc_tiling=False)
_gather_h = _make_gather2(H)
_scatter_msg = _make_scatter()


def _emb_body(zf, embp, out):
    zc = zf[...]
    io = lax.broadcasted_iota(jnp.int32, (BN, 128), 1).astype(jnp.float32)
    oh = jnp.where(io == zc, 1.0, 0.0)
    out[...] = jnp.dot(oh, embp[...], preferred_element_type=jnp.float32)


def _edge_body_first(ps, pd, hd, hs, offs, e1Wi, e1Wj, e1We, e1b, e2W, e2b,
                     fW1, fb1, fW2, fb2, cfW, oea, omsg):
    d = ps[...] - pd[...]
    dist = jnp.sqrt(jnp.sum(d * d, axis=1, keepdims=True) + 1e-9)
    delta = dist - offs[...]
    ea = jnp.exp(-12.5 * delta * delta)
    _edge_tail(ea, hd, hs, e1Wi, e1Wj, e1We, e1b, e2W, e2b,
               fW1, fb1, fW2, fb2, cfW, oea, omsg)


def _edge_body_rest(eain, hd, hs, e1Wi, e1Wj, e1We, e1b, e2W, e2b,
                    fW1, fb1, fW2, fb2, cfW, oea, omsg):
    _edge_tail(eain[...], hd, hs, e1Wi, e1Wj, e1We, e1b, e2W, e2b,
               fW1, fb1, fW2, fb2, cfW, oea, omsg)


def _edge_tail(ea, hd, hs, e1Wi, e1Wj, e1We, e1b, e2W, e2b,
               fW1, fb1, fW2, fb2, cfW, oea, omsg):
    dot = lambda a, b: jnp.dot(a, b, preferred_element_type=jnp.float32)
    xs = hs[...]
    u = _ssp(dot(hd[...], e1Wi[...]) + dot(xs, e1Wj[...])
             + dot(ea, e1We[...]) + e1b[...])
    ean = dot(u, e2W[...]) + e2b[...]
    oea[...] = ean
    t = _ssp(dot(ean, fW1[...]) + fb1[...])
    wf = _ssp(dot(t, fW2[...]) + fb2[...])
    omsg[...] = dot(xs, cfW[...]) * wf


def _node_body(h, ma, mb, sW1, sW2, out):
    dot = lambda a, b: jnp.dot(a, b, preferred_element_type=jnp.float32)
    msg = ma[...] + mb[...]
    out[...] = h[...] + dot(_ssp(dot(msg, sW1[...])), sW2[...])


def _final_body(h, ma, mb, sW1, sW2, oW1p, ob1p, oW2p, ob2p, b3, acc):
    dot = lambda a, b: jnp.dot(a, b, preferred_element_type=jnp.float32)
    msg = ma[...] + mb[...]
    h3 = h[...] + dot(_ssp(dot(msg, sW1[...])), sW2[...])
    v = _ssp(dot(h3, oW1p[...]) + ob1p[...])
    no = dot(v, oW2p[...]) + ob2p[...]
    bb = b3[...].reshape(1, BN)
    io = lax.broadcasted_iota(jnp.int32, (128, BN), 0)
    ohT = jnp.where(io == bb, 1.0, 0.0)
    part = dot(ohT, no)

    @pl.when(pl.program_id(0) == 0)
    def _():
        acc[...] = jnp.zeros_like(acc)

    acc[...] += part


def _full(shape):
    return pl.BlockSpec(shape, lambda i: (0, 0))


def _rows(shape):
    return pl.BlockSpec(shape, lambda i: (i, 0))


def _tc_emb(zf, embp):
    return pl.pallas_call(
        _emb_body,
        grid=(NB,),
        in_specs=[_rows((BN, 1)), _full((128, H))],
        out_specs=_rows((BN, H)),
        out_shape=jax.ShapeDtypeStruct((NP, H), jnp.float32),
    )(zf, embp)


def _tc_edge(first, ea_or_pos, hd, hs, wts, offs):
    e1Wi, e1Wj, e1We, e1b, e2W, e2b, fW1, fb1, fW2, fb2, cfW = wts
    wspecs = [_full((H, 2 * H)), _full((H, 2 * H)), _full((H, 2 * H)),
              _full((1, 2 * H)), _full((2 * H, H)), _full((1, H)),
              _full((H, H)), _full((1, H)), _full((H, H)), _full((1, H)),
              _full((H, H))]
    outsp = (_rows((BE, H)), _rows((BE, H)))
    outsh = (jax.ShapeDtypeStruct((E_PAD, H), jnp.float32),
             jax.ShapeDtypeStruct((E_PAD, H), jnp.float32))
    if first:
        ps, pd = ea_or_pos
        return pl.pallas_call(
            _edge_body_first,
            grid=(NEB,),
            in_specs=[_rows((BE, 16)), _rows((BE, 16)), _rows((BE, H)),
                      _rows((BE, H)), _full((1, 128))] + wspecs,
            out_specs=outsp,
            out_shape=outsh,
        )(ps, pd, hd, hs, offs, *wts)
    return pl.pallas_call(
        _edge_body_rest,
        grid=(NEB,),
        in_specs=[_rows((BE, H)), _rows((BE, H)), _rows((BE, H))] + wspecs,
        out_specs=outsp,
        out_shape=outsh,
    )(ea_or_pos, hd, hs, *wts)


def _tc_node(h, ma, mb, sW1, sW2):
    return pl.pallas_call(
        _node_body,
        grid=(NB,),
        in_specs=[_rows((BN, H))] * 3 + [_full((H, H))] * 2,
        out_specs=_rows((BN, H)),
        out_shape=jax.ShapeDtypeStruct((NP, H), jnp.float32),
    )(h, ma, mb, sW1, sW2)


def _tc_final(h, ma, mb, sW1, sW2, oW1p, ob1p, oW2p, ob2p, b3):
    return pl.pallas_call(
        _final_body,
        grid=(NB,),
        in_specs=[_rows((BN, H))] * 3 + [_full((H, H))] * 2
        + [_full((H, H)), _full((1, H)), _full((H, H)), _full((1, H)),
           pl.BlockSpec((1, 1, BN), lambda i: (i, 0, 0))],
        out_specs=_full((H, H)),
        out_shape=jax.ShapeDtypeStruct((H, H), jnp.float32),
    )(h, ma, mb, sW1, sW2, oW1p, ob1p, oW2p, ob2p, b3)


def kernel(pos, params, z, edge_index, batch):
    f32 = jnp.float32
    src = edge_index[0].astype(jnp.int32)
    dst = edge_index[1].astype(jnp.int32)
    src_p = jnp.pad(src, (0, E_PAD - E))
    dst_g = jnp.pad(dst, (0, E_PAD - E))
    dst_s = jnp.pad(dst, (0, E_PAD - E), constant_values=DUMMY)
    pos_p = jnp.pad(pos.astype(f32), ((0, 0), (0, 13)))
    zf = jnp.pad(z.astype(f32), (0, NP - N)).reshape(NP, 1)
    b3 = jnp.pad(batch.astype(jnp.int32), (0, NP - N),
                 constant_values=9999).reshape(NB, 1, BN)
    zblk = jnp.zeros((CH, H), f32)
    embp = jnp.pad(params["emb"].astype(f32), ((0, 128 - ZMAX), (0, 0)))
    offs = jnp.pad(jnp.linspace(0.0, 10.0 - 10.0 / G, G, dtype=f32),
                   (0, 128 - G)).reshape(1, 128)

    ps, pd = _gather_pos(pos_p, src_p, dst_g)
    h = _tc_emb(zf, embp)

    ea = None
    for l, lp in enumerate(params["layers"]):
        e1W = lp["e1W"].astype(f32)
        e1We = e1W[2 * H:]
        if l == 0:
            e1We = jnp.pad(e1We, ((0, 128 - G), (0, 0)))
        wts = (e1W[:H], e1W[H:2 * H], e1We, lp["e1b"].reshape(1, 2 * H),
               lp["e2W"], lp["e2b"].reshape(1, H), lp["fW1"],
               lp["fb1"].reshape(1, H), lp["fW2"], lp["fb2"].reshape(1, H),
               lp["cfW"])
        hs, hd = _gather_h(h, src_p, dst_g)
        ea, msg_e = _tc_edge(l == 0, (ps, pd) if l == 0 else ea, hd, hs,
                             wts, offs)
        parts = _scatter_msg(msg_e, dst_s, zblk)
        ma, mb = parts[:NP], parts[NP:]
        if l < 2:
            h = _tc_node(h, ma, mb, lp["sW1"], lp["sW2"])
        else:
            oW1p = jnp.pad(params["oW1"].astype(f32), ((0, 0), (0, H - H // 2)))
            ob1p = jnp.pad(params["ob1"].astype(f32), (0, H - H // 2)).reshape(1, H)
            oW2p = jnp.pad(jnp.broadcast_to(params["oW2"].astype(f32), (H // 2, H)),
                           ((0, H - H // 2), (0, 0)))
            ob2p = jnp.broadcast_to(params["ob2"].astype(f32), (1, H))
            acc = _tc_final(h, ma, mb, lp["sW1"], lp["sW2"],
                            oW1p, ob1p, oW2p, ob2p, b3)
    return acc[:NG, :1]
